# baseline (speedup 1.0000x reference)
import jax
import jax.numpy as jnp
from jax.experimental import pallas as pl
from jax.experimental.pallas import tpu as pltpu

EMBED_DIM = 16


def _body(idx_hbm, table_ref, out_ref, idx_s, blk_v, sem1, sem2):
    cp0 = pltpu.make_async_copy(idx_hbm, idx_s, sem1)
    cp0.start()
    cp0.wait()
    r = idx_s[0]
    base = pl.multiple_of((r // 128) * 128, 128)
    cp = pltpu.make_async_copy(table_ref.at[:, pl.ds(base, 128)], blk_v, sem2)
    cp.start()
    cp.wait()
    rolled = pltpu.roll(blk_v[...], -(r % 128), 1)
    out_ref[...] = jnp.swapaxes(rolled[:, :1], 0, 1)


def kernel(client_id, embed_table):
    idx = jnp.asarray(client_id, dtype=jnp.int32).reshape((1,))
    return pl.pallas_call(
        _body,
        in_specs=[
            pl.BlockSpec(memory_space=pl.ANY),
            pl.BlockSpec(memory_space=pl.ANY),
        ],
        out_shape=jax.ShapeDtypeStruct((1, EMBED_DIM), jnp.float32),
        scratch_shapes=[
            pltpu.SMEM((1,), jnp.int32),
            pltpu.VMEM((EMBED_DIM, 128), jnp.float32),
            pltpu.SemaphoreType.DMA,
            pltpu.SemaphoreType.DMA,
        ],
    )(idx, embed_table.T)


# speedup vs baseline: 1.2714x; 1.2714x over previous
import jax
import jax.numpy as jnp
from jax.experimental import pallas as pl
from jax.experimental.pallas import tpu as pltpu

EMBED_DIM = 16
LANES = 128


def _body(idx_ref, table_ref, out_ref):
    c = idx_ref[0] % LANES
    rolled = pltpu.roll(table_ref[...], -c, 1)
    out_ref[...] = jnp.swapaxes(rolled[:, :1], 0, 1)


def kernel(client_id, embed_table):
    idx = jnp.asarray(client_id, dtype=jnp.int32).reshape((1,))
    grid_spec = pltpu.PrefetchScalarGridSpec(
        num_scalar_prefetch=1,
        grid=(1,),
        in_specs=[
            pl.BlockSpec(
                (EMBED_DIM, LANES),
                lambda i, idx_ref: (0, idx_ref[0] // LANES),
            ),
        ],
        out_specs=pl.BlockSpec((1, EMBED_DIM), lambda i, idx_ref: (0, 0)),
    )
    return pl.pallas_call(
        _body,
        grid_spec=grid_spec,
        out_shape=jax.ShapeDtypeStruct((1, EMBED_DIM), jnp.float32),
    )(idx, embed_table.T)


# speedup vs baseline: 1.2772x; 1.0046x over previous
import jax
import jax.numpy as jnp
from jax.experimental import pallas as pl
from jax.experimental.pallas import tpu as pltpu

EMBED_DIM = 16
LANES = 128


def _body(idx_ref, hi_ref, lo_ref, out_ref):
    c = idx_ref[0] % LANES
    hi = pltpu.roll(hi_ref[...], -c, 1)[:, :1]
    lo = pltpu.roll(lo_ref[...], -c, 1)[:, :1]
    out_ref[...] = jnp.swapaxes(jnp.concatenate([hi, lo], 0), 0, 1)


def kernel(client_id, embed_table):
    idx = jnp.asarray(client_id, dtype=jnp.int32).reshape((1,))
    grid_spec = pltpu.PrefetchScalarGridSpec(
        num_scalar_prefetch=1,
        grid=(1,),
        in_specs=[
            pl.BlockSpec((8, LANES), lambda i, idx_ref: (0, idx_ref[0] // LANES)),
            pl.BlockSpec((8, LANES), lambda i, idx_ref: (1, idx_ref[0] // LANES)),
        ],
        out_specs=pl.BlockSpec((1, EMBED_DIM), lambda i, idx_ref: (0, 0)),
    )
    tt = embed_table.T
    return pl.pallas_call(
        _body,
        grid_spec=grid_spec,
        out_shape=jax.ShapeDtypeStruct((1, EMBED_DIM), jnp.float32),
    )(idx, tt, tt)
